# Initial kernel scaffold; baseline (speedup 1.0000x reference)
import jax
import jax.numpy as jnp
from jax.experimental import pallas as pl


def kernel(x):
    raise NotImplementedError("write your pallas kernel here")


# speedup vs baseline: 487.3656x; 487.3656x over previous
import numpy as np
import jax
import jax.numpy as jnp
from jax.experimental import pallas as pl
from jax.experimental.pallas import tpu as pltpu

_Q = 64
_T = 2048
_C = 128
_NLEV = 11


def _sort_stage(vs_ref, e, k_dyn, idx):
    j = 1 << e
    T, L = vs_ref.shape
    v = vs_ref[:]
    if j >= 8:
        g = T // (2 * j)
        vv = v.reshape(g, 2, j, L)
        a = vv[:, 0]
        b = vv[:, 1]
        mn = jnp.minimum(a, b)
        mx = jnp.maximum(a, b)
        gi = jax.lax.broadcasted_iota(jnp.int32, (g, 1, 1), 0)
        asc = ((gi * (2 * j)) & k_dyn) == 0
        lo = jnp.where(asc, mn, mx)
        hi = jnp.where(asc, mx, mn)
        vs_ref[:] = jnp.stack([lo, hi], axis=1).reshape(T, L)
    else:
        r = pltpu.roll(v, T - j, 0)
        l = pltpu.roll(v, j, 0)
        low = (idx & j) == 0
        asc = (idx & k_dyn) == 0
        partner = jnp.where(low, r, l)
        keep_min = low == asc
        mn = jnp.minimum(v, partner)
        mx = jnp.maximum(v, partner)
        vs_ref[:] = jnp.where(keep_min, mn, mx)


def _erfinv_f32(x):
    w = -jnp.log((1.0 - x) * (1.0 + x))
    in_core = w < 5.0
    wc = w - 2.5
    p1 = jnp.float32(2.81022636e-08)
    for c in (3.43273939e-07, -3.5233877e-06, -4.39150654e-06,
              0.00021858087, -0.00125372503, -0.00417768164,
              0.246640727, 1.50140941):
        p1 = jnp.float32(c) + p1 * wc
    wt = jnp.sqrt(jnp.maximum(w, 5.0)) - 3.0
    p2 = jnp.float32(-0.000200214257)
    for c in (0.000100950558, 0.00134934322, -0.00367342844,
              0.00573950773, -0.0076224613, 0.00943887047,
              1.00167406, 2.83297682):
        p2 = jnp.float32(c) + p2 * wt
    return jnp.where(in_core, p1, p2) * x


def _otnorm_kernel(x_ref, o_ref, vs_ref, q_ref):
    xb = x_ref[0]
    vs_ref[:] = xb
    idx = jax.lax.broadcasted_iota(jnp.int32, (_T, 1), 0)

    def phase(lk, carry):
        k_dyn = jnp.left_shift(jnp.int32(1), lk)
        for e in range(_NLEV - 1, -1, -1):
            @pl.when(e < lk)
            def _stage():
                _sort_stage(vs_ref, e, k_dyn, idx)
        return carry

    jax.lax.fori_loop(1, _NLEV + 1, phase, 0)

    sv = vs_ref[:].reshape(_Q, _T // _Q, _C)
    q = 0.5 * (sv[:, 15, :] + sv[:, 16, :])
    q_ref[:] = q

    big = jnp.float32(3.0e38)
    cnt = jnp.zeros_like(xb)
    x0 = jnp.full_like(xb, -big)
    x1 = jnp.full_like(xb, big)

    def body(k, carry):
        cnt, x0, x1 = carry
        qk = q_ref[pl.ds(k, 1), :]
        m = qk <= xb
        cnt = cnt + m.astype(jnp.float32)
        x0 = jnp.where(m, qk, x0)
        x1 = jnp.minimum(x1, jnp.where(m, big, qk))
        return cnt, x0, x1

    cnt, x0, x1 = jax.lax.fori_loop(0, _Q, body, (cnt, x0, x1))

    at_lo = cnt < 0.5
    at_hi = cnt > _Q - 0.5
    x0 = jnp.where(at_lo, q[0:1, :], x0)
    x1 = jnp.where(at_lo, q[1:2, :], x1)
    x0 = jnp.where(at_hi, q[_Q - 2:_Q - 1, :], x0)
    x1 = jnp.where(at_hi, q[_Q - 1:_Q, :], x1)
    idxf = jnp.clip(cnt, 1.0, float(_Q - 1))

    y0 = (idxf - 0.5) * (1.0 / _Q)
    slope = (1.0 / _Q) / jnp.maximum(x1 - x0, 1e-12)
    p = y0 + slope * (xb - x0)
    p = jnp.clip(p, 1e-6, 1.0 - 1e-6)
    z = _erfinv_f32(2.0 * p - 1.0) * np.sqrt(2.0).astype(np.float32)
    o_ref[0] = z


def kernel(x):
    B, T, C = x.shape
    assert T == _T and C == _C
    return pl.pallas_call(
        _otnorm_kernel,
        grid=(B,),
        in_specs=[pl.BlockSpec((1, T, C), lambda b: (b, 0, 0))],
        out_specs=pl.BlockSpec((1, T, C), lambda b: (b, 0, 0)),
        out_shape=jax.ShapeDtypeStruct((B, T, C), x.dtype),
        scratch_shapes=[
            pltpu.VMEM((_T, _C), jnp.float32),
            pltpu.VMEM((_Q, _C), jnp.float32),
        ],
    )(x)


# speedup vs baseline: 1775.6509x; 3.6434x over previous
import numpy as np
import jax
import jax.numpy as jnp
from jax.experimental import pallas as pl
from jax.experimental.pallas import tpu as pltpu

_Q = 64
_T = 2048
_C = 128
_R = _T // 8


def _bitonic_sort_permuted(v):
    T, L = v.shape
    sio = jax.lax.broadcasted_iota(jnp.int32, (T, 1), 0) & 7
    for lk in range(1, 12):
        k = 1 << lk
        for e in range(lk - 1, -1, -1):
            j = 1 << e
            if j <= 128:
                g = _R // (2 * j)
                vv = v.reshape(g, 2, j, 8, L)
                a = vv[:, 0]
                b = vv[:, 1]
                mn = jnp.minimum(a, b)
                mx = jnp.maximum(a, b)
                if k <= 128:
                    gi = jax.lax.broadcasted_iota(
                        jnp.int32, (g, 1, 1, 1), 0)
                    asc = ((gi * (2 * j)) & k) == 0
                elif k == 2048:
                    asc = None
                else:
                    kb = k // 256
                    si = jax.lax.broadcasted_iota(
                        jnp.int32, (1, 1, 8, 1), 2)
                    asc = (si & kb) == 0
                if asc is None:
                    lo, hi = mn, mx
                else:
                    lo = jnp.where(asc, mn, mx)
                    hi = jnp.where(asc, mx, mn)
                v = jnp.stack([lo, hi], axis=1).reshape(T, L)
            else:
                d = j // 256
                rr = pltpu.roll(v, T - d, 0)
                ll = pltpu.roll(v, d, 0)
                low = (sio & d) == 0
                partner = jnp.where(low, rr, ll)
                mn = jnp.minimum(v, partner)
                mx = jnp.maximum(v, partner)
                if k == 2048:
                    keep_min = low
                else:
                    kb = k // 256
                    asc = (sio & kb) == 0
                    keep_min = low == asc
                v = jnp.where(keep_min, mn, mx)
    return v


def _erfinv_f32(x):
    w = -jnp.log((1.0 - x) * (1.0 + x))
    in_core = w < 5.0
    wc = w - 2.5
    p1 = jnp.float32(2.81022636e-08)
    for c in (3.43273939e-07, -3.5233877e-06, -4.39150654e-06,
              0.00021858087, -0.00125372503, -0.00417768164,
              0.246640727, 1.50140941):
        p1 = jnp.float32(c) + p1 * wc
    wt = jnp.sqrt(jnp.maximum(w, 5.0)) - 3.0
    p2 = jnp.float32(-0.000200214257)
    for c in (0.000100950558, 0.00134934322, -0.00367342844,
              0.00573950773, -0.0076224613, 0.00943887047,
              1.00167406, 2.83297682):
        p2 = jnp.float32(c) + p2 * wt
    return jnp.where(in_core, p1, p2) * x


def _otnorm_kernel(x_ref, o_ref):
    xb = x_ref[0]
    s = _bitonic_sort_permuted(xb)
    sv = s.reshape(8, 32, 8, _C)
    a = sv[:, 15]
    b = sv[:, 16]
    qv = (0.5 * (a + b)).reshape(_Q, _C)

    def qrow(r):
        return qv[r:r + 1, :]

    big = jnp.float32(3.0e38)

    bI = jnp.zeros(xb.shape, jnp.int32)
    bmax = jnp.full_like(xb, -big)
    for j in range(7):
        br = qrow(56 + j)
        m = br <= xb
        bI = bI + m.astype(jnp.int32)
        bmax = jnp.where(m, br, bmax)
    b0 = (bI & 1) > 0
    b1 = (bI & 2) > 0
    b2 = (bI & 4) > 0

    fcnt = jnp.zeros_like(xb)
    x0 = jnp.full_like(xb, -big)
    x1 = jnp.full_like(xb, big)
    for i in range(8):
        r = [qrow(i * 8 + j) for j in range(8)]
        sa = jnp.where(b0, r[1], r[0])
        sb = jnp.where(b0, r[3], r[2])
        sc = jnp.where(b0, r[5], r[4])
        sd = jnp.where(b0, r[7], r[6])
        se = jnp.where(b1, sb, sa)
        sf = jnp.where(b1, sd, sc)
        qi = jnp.where(b2, sf, se)
        m = qi <= xb
        fcnt = fcnt + m.astype(jnp.float32)
        x0 = jnp.where(m, qi, x0)
        x1 = jnp.minimum(x1, jnp.where(m, big, qi))
    cnt = bI.astype(jnp.float32) * 8.0 + fcnt

    x0 = jnp.where(fcnt < 0.5, bmax, x0)

    at_lo = cnt < 0.5
    at_hi = cnt > _Q - 0.5
    x0 = jnp.where(at_lo, qrow(0), x0)
    x1 = jnp.where(at_lo, qrow(8), x1)
    x0 = jnp.where(at_hi, qrow(55), x0)
    x1 = jnp.where(at_hi, qrow(63), x1)
    idxf = jnp.clip(cnt, 1.0, float(_Q - 1))

    y0 = (idxf - 0.5) * (1.0 / _Q)
    slope = (1.0 / _Q) / jnp.maximum(x1 - x0, 1e-12)
    p = y0 + slope * (xb - x0)
    p = jnp.clip(p, 1e-6, 1.0 - 1e-6)
    z = _erfinv_f32(2.0 * p - 1.0) * np.sqrt(2.0).astype(np.float32)
    o_ref[0] = z


def kernel(x):
    B, T, C = x.shape
    assert T == _T and C == _C
    return pl.pallas_call(
        _otnorm_kernel,
        grid=(B,),
        in_specs=[pl.BlockSpec((1, T, C), lambda b: (b, 0, 0))],
        out_specs=pl.BlockSpec((1, T, C), lambda b: (b, 0, 0)),
        out_shape=jax.ShapeDtypeStruct((B, T, C), x.dtype),
    )(x)


# speedup vs baseline: 1824.1995x; 1.0273x over previous
import numpy as np
import jax
import jax.numpy as jnp
from jax.experimental import pallas as pl
from jax.experimental.pallas import tpu as pltpu

_Q = 64
_T = 2048
_C = 128
_R = _T // 8


def _row_asc_mask(g, k, group_rows):
    if k <= 128:
        gi = jax.lax.broadcasted_iota(jnp.int32, (g, 1, 1, 1), 0)
        return ((gi * group_rows) & k) == 0
    if k == 2048:
        return None
    kb = k // 256
    si = jax.lax.broadcasted_iota(jnp.int32, (1, 1, 8, 1), 2)
    return (si & kb) == 0


def _ce(a, b, asc):
    mn = jnp.minimum(a, b)
    mx = jnp.maximum(a, b)
    if asc is None:
        return mn, mx
    return jnp.where(asc, mn, mx), jnp.where(asc, mx, mn)


def _row_stage(v, e, k, L):
    j = 1 << e
    g = _R // (2 * j)
    vv = v.reshape(g, 2, j, 8, L)
    lo, hi = _ce(vv[:, 0], vv[:, 1], _row_asc_mask(g, k, 2 * j))
    return jnp.stack([lo, hi], axis=1).reshape(_T, L)


def _row_stage_pair(v, e2, k, L):
    j2 = 1 << e2
    g = _R // (4 * j2)
    vv = v.reshape(g, 2, 2, j2, 8, L)
    asc = _row_asc_mask(g, k, 4 * j2)
    q0 = vv[:, 0, 0]
    q1 = vv[:, 0, 1]
    q2 = vv[:, 1, 0]
    q3 = vv[:, 1, 1]
    b0, b2 = _ce(q0, q2, asc)
    b1, b3 = _ce(q1, q3, asc)
    c0, c1 = _ce(b0, b1, asc)
    c2, c3 = _ce(b2, b3, asc)
    return jnp.stack([c0, c1, c2, c3], axis=1).reshape(_T, L)


def _sublane_stage(v, e, k, sio):
    T = v.shape[0]
    d = (1 << e) // 256
    rr = pltpu.roll(v, T - d, 0)
    ll = pltpu.roll(v, d, 0)
    low = (sio & d) == 0
    partner = jnp.where(low, rr, ll)
    mn = jnp.minimum(v, partner)
    mx = jnp.maximum(v, partner)
    if k == 2048:
        keep_min = low
    else:
        asc = (sio & (k // 256)) == 0
        keep_min = low == asc
    return jnp.where(keep_min, mn, mx)


def _bitonic_sort_permuted(v):
    T, L = v.shape
    sio = jax.lax.broadcasted_iota(jnp.int32, (T, 1), 0) & 7
    for lk in range(1, 12):
        k = 1 << lk
        es = list(range(lk - 1, -1, -1))
        for e in es:
            if e > 7:
                v = _sublane_stage(v, e, k, sio)
        row_es = [e for e in es if e <= 7]
        i = 0
        while i < len(row_es):
            if i + 1 < len(row_es):
                v = _row_stage_pair(v, row_es[i + 1], k, L)
                i += 2
            else:
                v = _row_stage(v, row_es[i], k, L)
                i += 1
    return v


def _erfinv_f32(x):
    w = -jnp.log((1.0 - x) * (1.0 + x))
    in_core = w < 5.0
    wc = w - 2.5
    p1 = jnp.float32(2.81022636e-08)
    for c in (3.43273939e-07, -3.5233877e-06, -4.39150654e-06,
              0.00021858087, -0.00125372503, -0.00417768164,
              0.246640727, 1.50140941):
        p1 = jnp.float32(c) + p1 * wc
    wt = jnp.sqrt(jnp.maximum(w, 5.0)) - 3.0
    p2 = jnp.float32(-0.000200214257)
    for c in (0.000100950558, 0.00134934322, -0.00367342844,
              0.00573950773, -0.0076224613, 0.00943887047,
              1.00167406, 2.83297682):
        p2 = jnp.float32(c) + p2 * wt
    return jnp.where(in_core, p1, p2) * x


def _otnorm_kernel(x_ref, o_ref):
    xb = x_ref[0]
    s = _bitonic_sort_permuted(xb)
    sv = s.reshape(8, 32, 8, _C)
    a = sv[:, 15]
    b = sv[:, 16]
    qv = (0.5 * (a + b)).reshape(_Q, _C)

    def qrow(r):
        return qv[r:r + 1, :]

    big = jnp.float32(3.0e38)

    bI = jnp.zeros(xb.shape, jnp.int32)
    bmax = jnp.full_like(xb, -big)
    for j in range(7):
        br = qrow(56 + j)
        m = br <= xb
        bI = bI + m.astype(jnp.int32)
        bmax = jnp.where(m, br, bmax)
    b0 = (bI & 1) > 0
    b1 = (bI & 2) > 0
    b2 = (bI & 4) > 0

    fcnt = jnp.zeros_like(xb)
    x0 = jnp.full_like(xb, -big)
    x1 = jnp.full_like(xb, big)
    for i in range(8):
        r = [qrow(i * 8 + j) for j in range(8)]
        sa = jnp.where(b0, r[1], r[0])
        sb = jnp.where(b0, r[3], r[2])
        sc = jnp.where(b0, r[5], r[4])
        sd = jnp.where(b0, r[7], r[6])
        se = jnp.where(b1, sb, sa)
        sf = jnp.where(b1, sd, sc)
        qi = jnp.where(b2, sf, se)
        m = qi <= xb
        fcnt = fcnt + m.astype(jnp.float32)
        x0 = jnp.where(m, qi, x0)
        x1 = jnp.minimum(x1, jnp.where(m, big, qi))
    cnt = bI.astype(jnp.float32) * 8.0 + fcnt

    x0 = jnp.where(fcnt < 0.5, bmax, x0)

    at_lo = cnt < 0.5
    at_hi = cnt > _Q - 0.5
    x0 = jnp.where(at_lo, qrow(0), x0)
    x1 = jnp.where(at_lo, qrow(8), x1)
    x0 = jnp.where(at_hi, qrow(55), x0)
    x1 = jnp.where(at_hi, qrow(63), x1)
    idxf = jnp.clip(cnt, 1.0, float(_Q - 1))

    y0 = (idxf - 0.5) * (1.0 / _Q)
    slope = (1.0 / _Q) / jnp.maximum(x1 - x0, 1e-12)
    p = y0 + slope * (xb - x0)
    p = jnp.clip(p, 1e-6, 1.0 - 1e-6)
    z = _erfinv_f32(2.0 * p - 1.0) * np.sqrt(2.0).astype(np.float32)
    o_ref[0] = z


def kernel(x):
    B, T, C = x.shape
    assert T == _T and C == _C
    return pl.pallas_call(
        _otnorm_kernel,
        grid=(B,),
        in_specs=[pl.BlockSpec((1, T, C), lambda b: (b, 0, 0))],
        out_specs=pl.BlockSpec((1, T, C), lambda b: (b, 0, 0)),
        out_shape=jax.ShapeDtypeStruct((B, T, C), x.dtype),
    )(x)
